# Initial kernel scaffold; baseline (speedup 1.0000x reference)
import jax
import jax.numpy as jnp
from jax.experimental import pallas as pl


def kernel(x, edge_index, edge_weight, W, b):
    raise NotImplementedError("write your pallas kernel here")


# speedup vs baseline: 4.8228x; 4.8228x over previous
import functools

import jax
import jax.numpy as jnp
from jax import lax
from jax.experimental import pallas as pl
from jax.experimental.pallas import tpu as pltpu
from jax.experimental.pallas import tpu_sc as plsc

N_NODES = 10000
D = 128
N_EDGES = 320000

NUM_CORES = 2
NUM_SUBCORES = 16
NUM_WORKERS = NUM_CORES * NUM_SUBCORES
CHUNK = 128
K_CHUNKS = -(-N_EDGES // (NUM_WORKERS * CHUNK))
E_PAD = NUM_WORKERS * K_CHUNKS * CHUNK
ROWS_PER_TILE = 624
ROWS_TAIL = N_NODES - NUM_SUBCORES * ROWS_PER_TILE


def _sc_aggregate(x, col3, row3, ew3, zeros):
    mesh = plsc.VectorSubcoreMesh(core_axis_name="c", subcore_axis_name="s")

    @functools.partial(
        pl.kernel,
        mesh=mesh,
        out_type=jax.ShapeDtypeStruct((NUM_CORES, N_NODES, D), jnp.float32),
        scratch_types=[
            pltpu.VMEM((K_CHUNKS, CHUNK), jnp.int32),
            pltpu.VMEM((K_CHUNKS, CHUNK), jnp.int32),
            pltpu.VMEM((K_CHUNKS * CHUNK,), jnp.float32),
            pltpu.VMEM((CHUNK, D), jnp.float32),
            pltpu.VMEM_SHARED((N_NODES, D), jnp.float32),
            pltpu.SemaphoreType.DMA,
        ],
    )
    def agg(x_hbm, col_hbm, row_hbm, ew_hbm, zeros_hbm, out_hbm,
            col_v, row_v, ew_v, rows_v, acc, sem):
        c = lax.axis_index("c")
        s = lax.axis_index("s")
        wid = c * NUM_SUBCORES + s

        pltpu.sync_copy(zeros_hbm.at[pl.ds(s * ROWS_PER_TILE, ROWS_PER_TILE)],
                        acc.at[pl.ds(s * ROWS_PER_TILE, ROWS_PER_TILE)])

        @pl.when(s == NUM_SUBCORES - 1)
        def _zero_tail():
            base = NUM_SUBCORES * ROWS_PER_TILE
            pltpu.sync_copy(zeros_hbm.at[pl.ds(base, ROWS_TAIL)],
                            acc.at[pl.ds(base, ROWS_TAIL)])

        pltpu.sync_copy(col_hbm.at[wid], col_v)
        pltpu.sync_copy(row_hbm.at[wid], row_v)
        pltpu.sync_copy(ew_hbm.at[wid], ew_v)
        plsc.subcore_barrier()

        def chunk_body(k, carry):
            pltpu.async_copy(x_hbm.at[col_v.at[k]], rows_v, sem).wait()

            def group_body(g, carry2):
                w16 = ew_v[pl.ds(k * CHUNK + g * 16, 16)]
                for j in range(16):
                    wj = w16[j]
                    e = g * 16 + j
                    for t in range(D // 16):
                        sl = pl.ds(t * 16, 16)
                        rows_v[e, sl] = rows_v[e, sl] * wj
                return carry2

            lax.fori_loop(jnp.int32(0), jnp.int32(CHUNK // 16), group_body,
                          jnp.int32(0), unroll=False)

            pltpu.sync_copy(rows_v, acc.at[row_v.at[k]], add=True)
            return carry

        lax.fori_loop(jnp.int32(0), jnp.int32(K_CHUNKS), chunk_body,
                      jnp.int32(0), unroll=False)

        plsc.subcore_barrier()
        pltpu.sync_copy(acc.at[pl.ds(s * ROWS_PER_TILE, ROWS_PER_TILE)],
                        out_hbm.at[c, pl.ds(s * ROWS_PER_TILE, ROWS_PER_TILE)])

        @pl.when(s == NUM_SUBCORES - 1)
        def _export_tail():
            base = NUM_SUBCORES * ROWS_PER_TILE
            pltpu.sync_copy(acc.at[pl.ds(base, ROWS_TAIL)],
                            out_hbm.at[c, pl.ds(base, ROWS_TAIL)])

    return agg(x, col3, row3, ew3, zeros)


def _tc_linear(partials, Wt, b2):
    BM = 1000
    grid = (N_NODES // BM,)

    def body(p_ref, wt_ref, b_ref, o_ref):
        acc = p_ref[0] + p_ref[1]
        o_ref[...] = (
            jnp.dot(acc, wt_ref[...], preferred_element_type=jnp.float32)
            + b_ref[...]
        )

    return pl.pallas_call(
        body,
        grid=grid,
        in_specs=[
            pl.BlockSpec((NUM_CORES, BM, D), lambda i: (i * 0, i, i * 0)),
            pl.BlockSpec((D, D), lambda i: (i * 0, i * 0)),
            pl.BlockSpec((1, D), lambda i: (i * 0, i * 0)),
        ],
        out_specs=pl.BlockSpec((BM, D), lambda i: (i, i * 0)),
        out_shape=jax.ShapeDtypeStruct((N_NODES, D), jnp.float32),
    )(partials, Wt, b2)


def kernel(x, edge_index, edge_weight, W, b):
    x = x.astype(jnp.float32)
    row = edge_index[0].astype(jnp.int32)
    col = edge_index[1].astype(jnp.int32)
    ew = edge_weight.astype(jnp.float32)

    pad = E_PAD - N_EDGES
    row3 = jnp.concatenate([row, jnp.zeros((pad,), jnp.int32)]).reshape(
        NUM_WORKERS, K_CHUNKS, CHUNK)
    col3 = jnp.concatenate([col, jnp.zeros((pad,), jnp.int32)]).reshape(
        NUM_WORKERS, K_CHUNKS, CHUNK)
    ew3 = jnp.concatenate([ew, jnp.zeros((pad,), jnp.float32)]).reshape(
        NUM_WORKERS, K_CHUNKS * CHUNK)
    zeros = jnp.zeros((N_NODES, D), jnp.float32)

    partials = _sc_aggregate(x, col3, row3, ew3, zeros)

    Wt = W.astype(jnp.float32).T
    b2 = b.astype(jnp.float32).reshape(1, D)
    return _tc_linear(partials, Wt, b2)
